# baseline (speedup 1.0000x reference)
import functools

import numpy as np
import jax
import jax.numpy as jnp
from jax import lax
from jax.experimental import pallas as pl
from jax.experimental.pallas import tpu as pltpu
from jax.experimental.pallas import tpu_sc as plsc

_KS = 5
_SIGMA = 2.0
_HALF = _KS // 2

_NC = 2
_NS = 16
_NW = _NC * _NS
_L = 16
_CH = 64


def _gauss1d():
    d = np.arange(-_HALF, _HALF + 1, dtype=np.float64)
    g = np.exp(-(d * d) / (2.0 * _SIGMA * _SIGMA))
    g = g / g.sum()
    return [float(v) for v in g]


_G = _gauss1d()


def _roll(v, shift, axis):
    if shift == 0:
        return v
    return jnp.roll(v, shift, axis)


def _yblur_body(in_ref, out_ref):
    h = in_ref.shape[1]
    n = h - 2 * _HALF
    acc = _G[0] * (in_ref[:, pl.ds(0, n), :] + in_ref[:, pl.ds(4, n), :])
    acc += _G[1] * (in_ref[:, pl.ds(1, n), :] + in_ref[:, pl.ds(3, n), :])
    acc += _G[2] * in_ref[:, pl.ds(2, n), :]
    out_ref[:, pl.ds(_HALF, n), :] = acc


def _tr_body(in_ref, out_ref, t_ref):
    c, hb, w = in_ref.shape
    rows = hb * w
    for hl in range(hb):
        t_ref[pl.ds(hl * w, w), pl.ds(0, c)] = in_ref[:, hl, :].T
    t = t_ref[...]
    acc = _G[0] * (_roll(t, 2, 0) + _roll(t, -2, 0))
    acc += _G[1] * (_roll(t, 1, 0) + _roll(t, -1, 0))
    acc += _G[2] * t
    a0 = acc[:, 0:128].reshape(rows // 8, 1, 8, 128)
    a1 = acc[:, 128:256].reshape(rows // 8, 1, 8, 128)
    out_ref[...] = jnp.concatenate([a0, a1], axis=1).reshape(rows * 2, 128)


def _make_gather(hw, c, n):
    bpw = -(-(-(-n // _NW)) // 8) * 8
    n_chunks = -(-bpw // _CH)
    sizes = [_CH] * (n_chunks - 1)
    sizes.append(bpw - _CH * (n_chunks - 1))
    mesh = plsc.VectorSubcoreMesh(
        core_axis_name="c", subcore_axis_name="s",
        num_cores=_NC, num_subcores=_NS)

    @functools.partial(
        pl.kernel,
        mesh=mesh,
        compiler_params=pltpu.CompilerParams(use_tc_tiling_on_sc=False),
        out_type=jax.ShapeDtypeStruct((n, c), jnp.float32),
        scratch_types=[
            pltpu.VMEM((-(-bpw // _L) * _L,), jnp.int32),
            pltpu.VMEM((n_chunks, 2 * _CH), jnp.int32),
            pltpu.VMEM((2, 2 * _CH, 128), jnp.float32),
            pltpu.SemaphoreType.DMA,
            pltpu.SemaphoreType.DMA,
        ],
    )
    def gather_k(table_hbm, p_hbm, out_hbm, pv, offv, rows,
                 sem0, sem1):
        wid = lax.axis_index("s") * _NC + lax.axis_index("c")
        base = jnp.minimum(wid * bpw, jnp.int32(n - bpw))
        sems = (sem0, sem1)
        pltpu.sync_copy(p_hbm.at[pl.ds(base, bpw)], pv.at[pl.ds(0, bpw)])
        lo = jnp.int32(_HALF)
        hi = jnp.int32(511 - _HALF)

        def drain(j):
            sz = sizes[j]
            b = j % 2
            o = base + j * _CH
            pltpu.sync_copy(
                rows.at[b].at[pl.ds(0, sz)],
                out_hbm.at[pl.ds(o, sz), pl.ds(0, 128)])
            pltpu.sync_copy(
                rows.at[b].at[pl.ds(_CH, sz), pl.ds(0, c - 128)],
                out_hbm.at[pl.ds(o, sz), pl.ds(128, c - 128)])

        copies = [None] * n_chunks
        for j in range(n_chunks):
            for kk in range(_CH // _L):
                lane0 = j * _CH + kk * _L
                v = pv[pl.ds(lane0, _L)]
                xi = jnp.clip(v & jnp.int32(0xFFFF), lo, hi)
                yi = jnp.clip(v >> jnp.int32(16), lo, hi)
                q0 = (yi * jnp.int32(1024)
                      + (xi >> 3) * jnp.int32(16) + (xi & 7))
                offv[j, pl.ds(kk * _L, _L)] = q0
                offv[j, pl.ds(_CH + kk * _L, _L)] = q0 + jnp.int32(8)
            copies[j] = pltpu.async_copy(
                table_hbm.at[offv.at[j]], rows.at[j % 2], sems[j % 2])
            if j >= 1:
                copies[j - 1].wait()
                drain(j - 1)
        copies[n_chunks - 1].wait()
        drain(n_chunks - 1)

    return gather_k


def kernel(feature_map, keypoints):
    c, h, w = feature_map.shape
    n = keypoints.shape[0]

    cb = 4
    blurred = pl.pallas_call(
        _yblur_body,
        grid=(c // cb,),
        in_specs=[pl.BlockSpec((cb, h, w), lambda i: (i, 0, 0))],
        out_specs=pl.BlockSpec((cb, h, w), lambda i: (i, 0, 0)),
        out_shape=jax.ShapeDtypeStruct((c, h, w), jnp.float32),
    )(feature_map)

    hw = h * w
    cp = 256
    hb = 8
    table = pl.pallas_call(
        _tr_body,
        grid=(h // hb,),
        in_specs=[pl.BlockSpec((c, hb, w), lambda i: (0, i, 0))],
        out_specs=pl.BlockSpec((hb * w * 2, 128), lambda i: (i, 0)),
        out_shape=jax.ShapeDtypeStruct((hw * 2, 128), jnp.float32),
        scratch_shapes=[pltpu.VMEM((hb * w, cp), jnp.float32)],
    )(blurred)

    kp = keypoints.astype(jnp.int32)
    packed = (kp * jnp.array([1, 65536], jnp.int32)).sum(axis=1)
    return _make_gather(hw, c, n)(table, packed)


# speedup vs baseline: 1.1365x; 1.1365x over previous
import functools

import numpy as np
import jax
import jax.numpy as jnp
from jax import lax
from jax.experimental import pallas as pl
from jax.experimental.pallas import tpu as pltpu
from jax.experimental.pallas import tpu_sc as plsc

_KS = 5
_SIGMA = 2.0
_HALF = _KS // 2

_NC = 2
_NS = 16
_NW = _NC * _NS
_L = 16
_CH = 64


def _gauss1d():
    d = np.arange(-_HALF, _HALF + 1, dtype=np.float64)
    g = np.exp(-(d * d) / (2.0 * _SIGMA * _SIGMA))
    g = g / g.sum()
    return [float(v) for v in g]


_G = _gauss1d()


def _roll(v, shift, axis):
    if shift == 0:
        return v
    return jnp.roll(v, shift, axis)


def _yblur_body(in_ref, out_ref):
    h = in_ref.shape[1]
    n = h - 2 * _HALF
    acc = _G[0] * (in_ref[:, pl.ds(0, n), :] + in_ref[:, pl.ds(4, n), :])
    acc += _G[1] * (in_ref[:, pl.ds(1, n), :] + in_ref[:, pl.ds(3, n), :])
    acc += _G[2] * in_ref[:, pl.ds(2, n), :]
    out_ref[:, pl.ds(_HALF, n), :] = acc


def _tr_body(in_ref, out_ref, t_ref):
    c, hb, w = in_ref.shape
    rows = hb * w
    for hl in range(hb):
        t_ref[pl.ds(hl * w, w), pl.ds(0, c)] = in_ref[:, hl, :].T
    t = t_ref[...]
    acc = _G[0] * (_roll(t, 2, 0) + _roll(t, -2, 0))
    acc += _G[1] * (_roll(t, 1, 0) + _roll(t, -1, 0))
    acc += _G[2] * t
    a0 = acc[:, 0:128].reshape(rows // 8, 1, 8, 128)
    a1 = acc[:, 128:256].reshape(rows // 8, 1, 8, 128)
    out_ref[...] = jnp.concatenate([a0, a1], axis=1).reshape(rows * 2, 128)


def _out_tr_body(in_ref, out_ref):
    out_ref[...] = in_ref[...].T


def _make_gather(hw, c, n):
    bpw = -(-(-(-n // _NW)) // 8) * 8
    n_chunks = -(-bpw // _CH)
    sizes = [_CH] * (n_chunks - 1)
    sizes.append(bpw - _CH * (n_chunks - 1))
    mesh = plsc.VectorSubcoreMesh(
        core_axis_name="c", subcore_axis_name="s",
        num_cores=_NC, num_subcores=_NS)

    @functools.partial(
        pl.kernel,
        mesh=mesh,
        compiler_params=pltpu.CompilerParams(use_tc_tiling_on_sc=False),
        out_type=jax.ShapeDtypeStruct((n, c), jnp.float32),
        scratch_types=[
            pltpu.VMEM((-(-bpw // _L) * _L,), jnp.int32),
            pltpu.VMEM((n_chunks, 2 * _CH), jnp.int32),
            pltpu.VMEM((2, 2 * _CH, 128), jnp.float32),
            pltpu.SemaphoreType.DMA,
            pltpu.SemaphoreType.DMA,
        ],
    )
    def gather_k(table_hbm, p_hbm, out_hbm, pv, offv, rows,
                 sem0, sem1):
        wid = lax.axis_index("s") * _NC + lax.axis_index("c")
        base = jnp.minimum(wid * bpw, jnp.int32(n - bpw))
        sems = (sem0, sem1)
        pltpu.sync_copy(p_hbm.at[pl.ds(base, bpw)], pv.at[pl.ds(0, bpw)])
        lo = jnp.int32(_HALF)
        hi = jnp.int32(511 - _HALF)

        def drain(j):
            sz = sizes[j]
            b = j % 2
            o = base + j * _CH
            pltpu.sync_copy(
                rows.at[b].at[pl.ds(0, sz)],
                out_hbm.at[pl.ds(o, sz), pl.ds(0, 128)])
            pltpu.sync_copy(
                rows.at[b].at[pl.ds(_CH, sz), pl.ds(0, c - 128)],
                out_hbm.at[pl.ds(o, sz), pl.ds(128, c - 128)])

        copies = [None] * n_chunks
        for j in range(n_chunks):
            for kk in range(_CH // _L):
                lane0 = j * _CH + kk * _L
                v = pv[pl.ds(lane0, _L)]
                xi = jnp.clip(v & jnp.int32(0xFFFF), lo, hi)
                yi = jnp.clip(v >> jnp.int32(16), lo, hi)
                q0 = (yi * jnp.int32(1024)
                      + (xi >> 3) * jnp.int32(16) + (xi & 7))
                offv[j, pl.ds(kk * _L, _L)] = q0
                offv[j, pl.ds(_CH + kk * _L, _L)] = q0 + jnp.int32(8)
            copies[j] = pltpu.async_copy(
                table_hbm.at[offv.at[j]], rows.at[j % 2], sems[j % 2])
            if j >= 1:
                copies[j - 1].wait()
                drain(j - 1)
        copies[n_chunks - 1].wait()
        drain(n_chunks - 1)

    return gather_k


def kernel(feature_map, keypoints):
    c, h, w = feature_map.shape
    n = keypoints.shape[0]

    cb = 4
    blurred = pl.pallas_call(
        _yblur_body,
        grid=(c // cb,),
        in_specs=[pl.BlockSpec((cb, h, w), lambda i: (i, 0, 0))],
        out_specs=pl.BlockSpec((cb, h, w), lambda i: (i, 0, 0)),
        out_shape=jax.ShapeDtypeStruct((c, h, w), jnp.float32),
    )(feature_map)

    hw = h * w
    cp = 256
    hb = 8
    table = pl.pallas_call(
        _tr_body,
        grid=(h // hb,),
        in_specs=[pl.BlockSpec((c, hb, w), lambda i: (0, i, 0))],
        out_specs=pl.BlockSpec((hb * w * 2, 128), lambda i: (i, 0)),
        out_shape=jax.ShapeDtypeStruct((hw * 2, 128), jnp.float32),
        scratch_shapes=[pltpu.VMEM((hb * w, cp), jnp.float32)],
    )(blurred)

    kp = keypoints.astype(jnp.int32)
    packed = (kp * jnp.array([1, 65536], jnp.int32)).sum(axis=1)
    out_nc = _make_gather(hw, c, n)(table, packed)

    out_cn = pl.pallas_call(
        _out_tr_body,
        in_specs=[pl.BlockSpec((n, c), lambda: (0, 0))],
        out_specs=pl.BlockSpec((c, n), lambda: (0, 0)),
        out_shape=jax.ShapeDtypeStruct((c, n), jnp.float32),
    )(out_nc)
    return out_cn.T


# speedup vs baseline: 1.2253x; 1.0781x over previous
import functools

import numpy as np
import jax
import jax.numpy as jnp
from jax import lax
from jax.experimental import pallas as pl
from jax.experimental.pallas import tpu as pltpu
from jax.experimental.pallas import tpu_sc as plsc

_KS = 5
_SIGMA = 2.0
_HALF = _KS // 2

_NC = 2
_NS = 16
_NW = _NC * _NS
_L = 16
_CH = 64


def _gauss1d():
    d = np.arange(-_HALF, _HALF + 1, dtype=np.float64)
    g = np.exp(-(d * d) / (2.0 * _SIGMA * _SIGMA))
    g = g / g.sum()
    return [float(v) for v in g]


_G = _gauss1d()


def _roll(v, shift, axis):
    if shift == 0:
        return v
    return jnp.roll(v, shift, axis)


def _yblur_body(in_ref, out_ref):
    h = in_ref.shape[1]
    n = h - 2 * _HALF
    acc = _G[0] * (in_ref[:, pl.ds(0, n), :] + in_ref[:, pl.ds(4, n), :])
    acc += _G[1] * (in_ref[:, pl.ds(1, n), :] + in_ref[:, pl.ds(3, n), :])
    acc += _G[2] * in_ref[:, pl.ds(2, n), :]
    out_ref[:, pl.ds(_HALF, n), :] = acc


def _tr_body(in_ref, out_ref, t_ref):
    c, hb, w = in_ref.shape
    rows = hb * w
    for hl in range(hb):
        t_ref[pl.ds(hl * w, w), pl.ds(0, c)] = in_ref[:, hl, :].T
    t = t_ref[...]
    acc = _G[0] * (_roll(t, 2, 0) + _roll(t, -2, 0))
    acc += _G[1] * (_roll(t, 1, 0) + _roll(t, -1, 0))
    acc += _G[2] * t
    a0 = acc[:, 0:128].reshape(rows // 8, 1, 8, 128)
    a1 = acc[:, 128:256].reshape(rows // 8, 1, 8, 128)
    out_ref[...] = jnp.concatenate([a0, a1], axis=1).reshape(rows * 2, 128)


def _out_tr_body(in_ref, out_ref):
    out_ref[...] = in_ref[...].T


def _make_gather(hw, c, n):
    bpw = -(-(-(-n // _NW)) // 8) * 8
    n_chunks = -(-bpw // _CH)
    sizes = [_CH] * (n_chunks - 1)
    sizes.append(bpw - _CH * (n_chunks - 1))
    mesh = plsc.VectorSubcoreMesh(
        core_axis_name="c", subcore_axis_name="s",
        num_cores=_NC, num_subcores=_NS)

    @functools.partial(
        pl.kernel,
        mesh=mesh,
        compiler_params=pltpu.CompilerParams(use_tc_tiling_on_sc=False),
        out_type=jax.ShapeDtypeStruct((n, c), jnp.float32),
        scratch_types=[
            pltpu.VMEM((-(-bpw // _L) * _L,), jnp.int32),
            pltpu.VMEM((n_chunks, 2 * _CH), jnp.int32),
            pltpu.VMEM((2, 2 * _CH, 128), jnp.float32),
            pltpu.SemaphoreType.DMA,
            pltpu.SemaphoreType.DMA,
        ],
    )
    def gather_k(table_hbm, p_hbm, out_hbm, pv, offv, rows,
                 sem0, sem1):
        wid = lax.axis_index("s") * _NC + lax.axis_index("c")
        base = jnp.minimum(wid * bpw, jnp.int32(n - bpw))
        sems = (sem0, sem1)
        pltpu.sync_copy(p_hbm.at[pl.ds(base, bpw)], pv.at[pl.ds(0, bpw)])
        lo = jnp.int32(_HALF)
        hi = jnp.int32(511 - _HALF)

        def drain(j):
            sz = sizes[j]
            b = j % 2
            o = base + j * _CH
            pltpu.sync_copy(
                rows.at[b].at[pl.ds(0, sz)],
                out_hbm.at[pl.ds(o, sz), pl.ds(0, 128)])
            pltpu.sync_copy(
                rows.at[b].at[pl.ds(_CH, sz), pl.ds(0, c - 128)],
                out_hbm.at[pl.ds(o, sz), pl.ds(128, c - 128)])

        copies = [None] * n_chunks
        for j in range(n_chunks):
            for kk in range(_CH // _L):
                lane0 = j * _CH + kk * _L
                v = pv[pl.ds(lane0, _L)]
                xi = jnp.clip(v & jnp.int32(0xFFFF), lo, hi)
                yi = jnp.clip(v >> jnp.int32(16), lo, hi)
                q0 = (yi * jnp.int32(1024)
                      + (xi >> 3) * jnp.int32(16) + (xi & 7))
                offv[j, pl.ds(kk * _L, _L)] = q0
                offv[j, pl.ds(_CH + kk * _L, _L)] = q0 + jnp.int32(8)
            copies[j] = pltpu.async_copy(
                table_hbm.at[offv.at[j]], rows.at[j % 2], sems[j % 2])
            if j >= 1:
                copies[j - 1].wait()
                drain(j - 1)
        copies[n_chunks - 1].wait()
        drain(n_chunks - 1)

    return gather_k


def kernel(feature_map, keypoints):
    c, h, w = feature_map.shape
    n = keypoints.shape[0]

    cb = 8
    blurred = pl.pallas_call(
        _yblur_body,
        grid=(c // cb,),
        in_specs=[pl.BlockSpec((cb, h, w), lambda i: (i, 0, 0))],
        out_specs=pl.BlockSpec((cb, h, w), lambda i: (i, 0, 0)),
        out_shape=jax.ShapeDtypeStruct((c, h, w), jnp.float32),
    )(feature_map)

    hw = h * w
    cp = 256
    hb = 16
    table = pl.pallas_call(
        _tr_body,
        grid=(h // hb,),
        in_specs=[pl.BlockSpec((c, hb, w), lambda i: (0, i, 0))],
        out_specs=pl.BlockSpec((hb * w * 2, 128), lambda i: (i, 0)),
        out_shape=jax.ShapeDtypeStruct((hw * 2, 128), jnp.float32),
        scratch_shapes=[pltpu.VMEM((hb * w, cp), jnp.float32)],
    )(blurred)

    kp = keypoints.astype(jnp.int32)
    packed = (kp * jnp.array([1, 65536], jnp.int32)).sum(axis=1)
    out_nc = _make_gather(hw, c, n)(table, packed)

    out_cn = pl.pallas_call(
        _out_tr_body,
        in_specs=[pl.BlockSpec((n, c), lambda: (0, 0))],
        out_specs=pl.BlockSpec((c, n), lambda: (0, 0)),
        out_shape=jax.ShapeDtypeStruct((c, n), jnp.float32),
    )(out_nc)
    return out_cn.T
